# baseline (speedup 1.0000x reference)
import jax
import jax.numpy as jnp
from jax import lax
from jax.experimental import pallas as pl
from jax.experimental.pallas import tpu as pltpu
import functools

_B, _H, _W, _CIN = 32, 64, 64, 16
_WIDTH, _M1, _M2 = 32, 12, 12
_PACK = 4
_M = _B * _H * _W
_MP = _M // _PACK
_TM = 4096
_INV63 = 1.0 / 63.0

_CPARAMS = pltpu.CompilerParams(dimension_semantics=("parallel",),
                                vmem_limit_bytes=48 * 1024 * 1024)


def _gelu(v):
    return 0.5 * v * (1.0 + jnp.tanh(0.7978845608028654
                                     * (v + 0.044715 * v * v * v)))


def _fc0_kernel(x_ref, w_ref, g_ref, o_ref):
    i = pl.program_id(0)
    y = jnp.dot(x_ref[...], w_ref[...], preferred_element_type=jnp.float32)
    tm = y.shape[0]
    r = lax.broadcasted_iota(jnp.int32, (tm, 1, 1), 0) + i * tm
    j = lax.broadcasted_iota(jnp.int32, (tm, 4, 1), 1)
    gx = ((r & 15) * 4 + j).astype(jnp.float32) * _INV63
    gy = ((r >> 4) & 63).astype(jnp.float32) * _INV63
    g = g_ref[...]
    add = (gx * g[0].reshape(1, 1, 32) + gy * g[1].reshape(1, 1, 32)
           + g[2].reshape(1, 1, 32))
    o_ref[...] = y + add.reshape(tm, 128)


def _res_kernel(x_ref, s_ref, w_ref, b_ref, o_ref):
    y = (s_ref[...]
         + jnp.dot(x_ref[...], w_ref[...], preferred_element_type=jnp.float32)
         + b_ref[...])
    o_ref[...] = _gelu(y)


def _spec_kernel(x_ref, w_ref, o_ref):
    o_ref[...] = jnp.dot(x_ref[...], w_ref[...],
                         preferred_element_type=jnp.float32)


def _tail_kernel(x_ref, s_ref, w3_ref, b3_ref, w1_ref, b1_ref,
                 w2_ref, b2_ref, o_ref):
    h = (s_ref[...]
         + jnp.dot(x_ref[...], w3_ref[...], preferred_element_type=jnp.float32)
         + b3_ref[...])
    t = _gelu(jnp.dot(h, w1_ref[...], preferred_element_type=jnp.float32)
              + b1_ref[...])
    o_ref[...] = (jnp.dot(t, w2_ref[...], preferred_element_type=jnp.float32)
                  + b2_ref[...])


def _row_call(body, nin_specs, out_cols, *args):
    grid = (_MP // _TM,)
    return pl.pallas_call(
        body,
        out_shape=jax.ShapeDtypeStruct((_MP, out_cols), jnp.float32),
        grid=grid,
        in_specs=nin_specs,
        out_specs=pl.BlockSpec((_TM, out_cols), lambda i: (i, 0)),
        compiler_params=_CPARAMS,
    )(*args)


def _rowspec(cols):
    return pl.BlockSpec((_TM, cols), lambda i: (i, 0))


def _wspec(rows, cols):
    return pl.BlockSpec((rows, cols), lambda i: (0, 0))


def _spectral(h, spec_w):
    ng, kg, _ = spec_w.shape
    x_ft = jnp.fft.rfft2(h, axes=(1, 2))
    kept = jnp.concatenate([x_ft[:, :_M1, :_M2, :],
                            x_ft[:, _H - _M1:, :_M2, :]], axis=1)
    kept = kept.reshape(_B, 2 * _M1 * _M2, _WIDTH)
    xp = jnp.concatenate([jnp.real(kept), jnp.imag(kept)], axis=-1)
    xp = xp.reshape(_B, -1)
    out = pl.pallas_call(
        _spec_kernel,
        out_shape=jax.ShapeDtypeStruct((_B, ng * kg), jnp.float32),
        grid=(ng,),
        in_specs=[pl.BlockSpec((_B, kg), lambda i: (0, i)),
                  pl.BlockSpec((None, kg, kg), lambda i: (i, 0, 0))],
        out_specs=pl.BlockSpec((_B, kg), lambda i: (0, i)),
        compiler_params=_CPARAMS,
    )(xp, spec_w)
    out = out.reshape(_B, 2 * _M1 * _M2, 2 * _WIDTH)
    oc = lax.complex(out[..., :_WIDTH], out[..., _WIDTH:])
    oc = oc.reshape(_B, 2 * _M1, _M2, _WIDTH)
    full = jnp.zeros((_B, _H, _W // 2 + 1, _WIDTH), jnp.complex64)
    full = full.at[:, :_M1, :_M2, :].set(oc[:, :_M1])
    full = full.at[:, _H - _M1:, :_M2, :].set(oc[:, _M1:])
    return jnp.fft.irfft2(full, s=(_H, _W), axes=(1, 2))


def kernel(fc0_w, fc0_b, w0_w, w0_b, spec0_w, w1_w, w1_b, spec1_w,
           w2_w, w2_b, spec2_w, w3_w, w3_b, spec3_w,
           fc1_w, fc1_b, fc2_w, fc2_b, x):
    w16 = fc0_w[:_CIN, :_WIDTH]
    w64 = jnp.kron(jnp.eye(_PACK, dtype=w16.dtype), w16)
    gpar = jnp.stack([fc0_w[_CIN, :_WIDTH], fc0_w[_CIN + 1, :_WIDTH],
                      fc0_b[:_WIDTH], jnp.zeros((_WIDTH,), jnp.float32)])

    xp = x.reshape(_MP, _PACK * _CIN)
    h = _row_call(_fc0_kernel,
                  [_rowspec(_PACK * _CIN), _wspec(64, 128),
                   pl.BlockSpec((4, 32), lambda i: (0, 0))],
                  128, xp, w64, gpar)

    for wk, bk, sk in ((w0_w, w0_b, spec0_w), (w1_w, w1_b, spec1_w),
                       (w2_w, w2_b, spec2_w)):
        spec = _spectral(h.reshape(_B, _H, _W, _WIDTH), sk)
        h = _row_call(_res_kernel,
                      [_rowspec(128), _rowspec(128), _wspec(128, 128),
                       pl.BlockSpec((1, 128), lambda i: (0, 0))],
                      128, h, spec.reshape(_MP, 128), wk, bk.reshape(1, 128))

    spec = _spectral(h.reshape(_B, _H, _W, _WIDTH), spec3_w)
    n1 = fc1_w.shape[1]
    out = _row_call(_tail_kernel,
                    [_rowspec(128), _rowspec(128), _wspec(128, 128),
                     pl.BlockSpec((1, 128), lambda i: (0, 0)),
                     _wspec(128, n1), pl.BlockSpec((1, n1), lambda i: (0, 0)),
                     _wspec(n1, _PACK), pl.BlockSpec((1, _PACK), lambda i: (0, 0))],
                    _PACK, h, spec.reshape(_MP, 128), w3_w,
                    w3_b.reshape(1, 128), fc1_w, fc1_b.reshape(1, n1),
                    fc2_w, fc2_b.reshape(1, _PACK))
    return out.reshape(_B, _H, _W, 1)


# speedup vs baseline: 4.9197x; 4.9197x over previous
import jax
import jax.numpy as jnp
from jax import lax
from jax.experimental import pallas as pl
from jax.experimental.pallas import tpu as pltpu

_B, _H, _W, _CIN = 32, 64, 64, 16
_WIDTH, _M1, _M2 = 32, 12, 12
_PACK = 4
_M = _B * _H * _W
_MP = _M // _PACK
_TM = 4096
_GB = 4
_INV63 = 1.0 / 63.0
_WC = _W * _WIDTH
_NK = 2 * _M1
_KW = _M2
_LF = _KW * _WIDTH

_CPARAMS = pltpu.CompilerParams(dimension_semantics=("parallel",),
                                vmem_limit_bytes=48 * 1024 * 1024)


def _gelu(v):
    return 0.5 * v * (1.0 + jnp.tanh(0.7978845608028654
                                     * (v + 0.044715 * v * v * v)))


def _fc0_kernel(x_ref, w_ref, g_ref, o_ref):
    i = pl.program_id(0)
    y = jnp.dot(x_ref[...], w_ref[...], preferred_element_type=jnp.float32)
    tm = y.shape[0]
    r = lax.broadcasted_iota(jnp.int32, (tm, 1, 1), 0) + i * tm
    j = lax.broadcasted_iota(jnp.int32, (tm, 4, 1), 1)
    gx = ((r & 15) * 4 + j).astype(jnp.float32) * _INV63
    gy = ((r >> 4) & 63).astype(jnp.float32) * _INV63
    g = g_ref[...]
    add = (gx * g[0].reshape(1, 1, 32) + gy * g[1].reshape(1, 1, 32)
           + g[2].reshape(1, 1, 32))
    o_ref[...] = y + add.reshape(tm, 128)


def _res_kernel(x_ref, s_ref, w_ref, b_ref, o_ref):
    y = (s_ref[...]
         + jnp.dot(x_ref[...], w_ref[...], preferred_element_type=jnp.float32)
         + b_ref[...])
    o_ref[...] = _gelu(y)


def _spec_kernel(x_ref, w_ref, o_ref):
    o_ref[...] = jnp.dot(x_ref[...], w_ref[...],
                         preferred_element_type=jnp.float32)


def _fwd_dft_kernel(x_ref, a1_ref, kb_ref, o_ref):
    x = x_ref[...].reshape(_GB * _H, _WC)
    r = jnp.dot(a1_ref[...], x, preferred_element_type=jnp.float32)
    t = jnp.dot(r, kb_ref[...], preferred_element_type=jnp.float32)
    rows = []
    for b in range(_GB):
        top = t[b * 2 * _NK: b * 2 * _NK + _NK]
        bot = t[b * 2 * _NK + _NK: (b + 1) * 2 * _NK]
        rows.append(top[:, :_LF] - bot[:, _LF:])
        rows.append(top[:, _LF:] + bot[:, :_LF])
    o_ref[...] = jnp.concatenate(rows, axis=0).reshape(_GB, 2 * _NK, _LF)


def _inv_dft_kernel(x_ref, a2_ref, ki_ref, o_ref):
    x = x_ref[...].reshape(_GB * 2 * _NK, _LF)
    t = jnp.dot(a2_ref[...], x, preferred_element_type=jnp.float32)
    rows = []
    for b in range(_GB):
        blk = t[b * 2 * _H: (b + 1) * 2 * _H]
        rows.append(jnp.concatenate([blk[:_H], blk[_H:]], axis=1))
    y_in = jnp.concatenate(rows, axis=0)
    y = jnp.dot(y_in, ki_ref[...], preferred_element_type=jnp.float32)
    o_ref[...] = y.reshape(_GB, _H, _WC)


def _tail_kernel(x_ref, s_ref, w3_ref, b3_ref, w1_ref, b1_ref,
                 w2_ref, b2_ref, o_ref):
    h = (s_ref[...]
         + jnp.dot(x_ref[...], w3_ref[...], preferred_element_type=jnp.float32)
         + b3_ref[...])
    t = _gelu(jnp.dot(h, w1_ref[...], preferred_element_type=jnp.float32)
              + b1_ref[...])
    o_ref[...] = (jnp.dot(t, w2_ref[...], preferred_element_type=jnp.float32)
                  + b2_ref[...])


def _row_call(body, in_specs, out_cols, *args):
    return pl.pallas_call(
        body,
        out_shape=jax.ShapeDtypeStruct((_MP, out_cols), jnp.float32),
        grid=(_MP // _TM,),
        in_specs=in_specs,
        out_specs=pl.BlockSpec((_TM, out_cols), lambda i: (i, 0)),
        compiler_params=_CPARAMS,
    )(*args)


def _rowspec(cols):
    return pl.BlockSpec((_TM, cols), lambda i: (i, 0))


def _wspec(rows, cols):
    return pl.BlockSpec((rows, cols), lambda i: (0, 0))


def _dft_consts():
    th = 2.0 * jnp.pi / 64.0
    hh = jnp.arange(64, dtype=jnp.float32)
    kk = jnp.concatenate([jnp.arange(12, dtype=jnp.float32),
                          jnp.arange(52, 64, dtype=jnp.float32)])
    kw = jnp.arange(12, dtype=jnp.float32)
    eye_c = jnp.eye(_WIDTH, dtype=jnp.float32)
    eye_g = jnp.eye(_GB, dtype=jnp.float32)
    a1 = th * kk[:, None] * hh[None, :]
    a1 = jnp.concatenate([jnp.cos(a1), -jnp.sin(a1)], axis=0)
    a1 = jnp.kron(eye_g, a1)
    aw = th * hh[:, None] * kw[None, :]
    kbig = jnp.concatenate([jnp.kron(jnp.cos(aw), eye_c),
                            jnp.kron(-jnp.sin(aw), eye_c)], axis=1)
    a2a = th * hh[:, None] * kk[None, :]
    ar, ai = jnp.cos(a2a) / 64.0, jnp.sin(a2a) / 64.0
    a2 = jnp.concatenate(
        [jnp.concatenate([ar, -ai], axis=1),
         jnp.concatenate([ai, ar], axis=1)], axis=0)
    a2 = jnp.kron(eye_g, a2)
    gk = jnp.where(kw == 0, 1.0, 2.0)
    awi = th * kw[:, None] * hh[None, :]
    kinv = jnp.concatenate(
        [jnp.kron(gk[:, None] * jnp.cos(awi) / 64.0, eye_c),
         jnp.kron(gk[:, None] * -jnp.sin(awi) / 64.0, eye_c)], axis=0)
    return a1, kbig, a2, kinv


def _spectral(h, spec_w, a1, kbig, a2, kinv):
    ng, kg, _ = spec_w.shape
    modes = pl.pallas_call(
        _fwd_dft_kernel,
        out_shape=jax.ShapeDtypeStruct((_B, 2 * _NK, _LF), jnp.float32),
        grid=(_B // _GB,),
        in_specs=[pl.BlockSpec((_GB, _H, _WC), lambda i: (i, 0, 0)),
                  _wspec(*a1.shape), _wspec(*kbig.shape)],
        out_specs=pl.BlockSpec((_GB, 2 * _NK, _LF), lambda i: (i, 0, 0)),
        compiler_params=_CPARAMS,
    )(h.reshape(_B, _H, _WC), a1, kbig)
    re = modes[:, :_NK].reshape(_B, _NK, _KW, _WIDTH)
    im = modes[:, _NK:].reshape(_B, _NK, _KW, _WIDTH)
    xp = jnp.concatenate([re, im], axis=-1).reshape(_B, ng * kg)
    mixed = pl.pallas_call(
        _spec_kernel,
        out_shape=jax.ShapeDtypeStruct((_B, ng * kg), jnp.float32),
        grid=(ng,),
        in_specs=[pl.BlockSpec((_B, kg), lambda i: (0, i)),
                  pl.BlockSpec((None, kg, kg), lambda i: (i, 0, 0))],
        out_specs=pl.BlockSpec((_B, kg), lambda i: (0, i)),
        compiler_params=_CPARAMS,
    )(xp, spec_w)
    mixed = mixed.reshape(_B, _NK * _KW, 2 * _WIDTH)
    rec = mixed[..., :_WIDTH].reshape(_B, _NK, _LF)
    imc = mixed[..., _WIDTH:].reshape(_B, _NK, _LF)
    x3 = jnp.concatenate([rec, imc], axis=1)
    out = pl.pallas_call(
        _inv_dft_kernel,
        out_shape=jax.ShapeDtypeStruct((_B, _H, _WC), jnp.float32),
        grid=(_B // _GB,),
        in_specs=[pl.BlockSpec((_GB, 2 * _NK, _LF), lambda i: (i, 0, 0)),
                  _wspec(*a2.shape), _wspec(*kinv.shape)],
        out_specs=pl.BlockSpec((_GB, _H, _WC), lambda i: (i, 0, 0)),
        compiler_params=_CPARAMS,
    )(x3, a2, kinv)
    return out.reshape(_MP, _PACK * _WIDTH)


def kernel(fc0_w, fc0_b, w0_w, w0_b, spec0_w, w1_w, w1_b, spec1_w,
           w2_w, w2_b, spec2_w, w3_w, w3_b, spec3_w,
           fc1_w, fc1_b, fc2_w, fc2_b, x):
    w16 = fc0_w[:_CIN, :_WIDTH]
    w64 = jnp.kron(jnp.eye(_PACK, dtype=w16.dtype), w16)
    gpar = jnp.stack([fc0_w[_CIN, :_WIDTH], fc0_w[_CIN + 1, :_WIDTH],
                      fc0_b[:_WIDTH], jnp.zeros((_WIDTH,), jnp.float32)])
    a1, kbig, a2, kinv = _dft_consts()

    xp = x.reshape(_MP, _PACK * _CIN)
    h = _row_call(_fc0_kernel,
                  [_rowspec(_PACK * _CIN), _wspec(64, 128),
                   pl.BlockSpec((4, 32), lambda i: (0, 0))],
                  128, xp, w64, gpar)

    for wk, bk, sk in ((w0_w, w0_b, spec0_w), (w1_w, w1_b, spec1_w),
                       (w2_w, w2_b, spec2_w)):
        spec = _spectral(h, sk, a1, kbig, a2, kinv)
        h = _row_call(_res_kernel,
                      [_rowspec(128), _rowspec(128), _wspec(128, 128),
                       pl.BlockSpec((1, 128), lambda i: (0, 0))],
                      128, h, spec, wk, bk.reshape(1, 128))

    spec = _spectral(h, spec3_w, a1, kbig, a2, kinv)
    n1 = fc1_w.shape[1]
    out = _row_call(_tail_kernel,
                    [_rowspec(128), _rowspec(128), _wspec(128, 128),
                     pl.BlockSpec((1, 128), lambda i: (0, 0)),
                     _wspec(128, n1), pl.BlockSpec((1, n1), lambda i: (0, 0)),
                     _wspec(n1, _PACK), pl.BlockSpec((1, _PACK), lambda i: (0, 0))],
                    _PACK, h, spec, w3_w, w3_b.reshape(1, 128),
                    fc1_w, fc1_b.reshape(1, n1), fc2_w, fc2_b.reshape(1, _PACK))
    return out.reshape(_B, _H, _W, 1)


# speedup vs baseline: 4.9529x; 1.0068x over previous
import jax
import jax.numpy as jnp
from jax import lax
from jax.experimental import pallas as pl
from jax.experimental.pallas import tpu as pltpu

_B, _H, _W, _CIN = 32, 64, 64, 16
_WIDTH, _M1, _M2 = 32, 12, 12
_PACK = 4
_M = _B * _H * _W
_MP = _M // _PACK
_TM = 4096
_GB = 4
_INV63 = 1.0 / 63.0
_WC = _W * _WIDTH
_NK = 2 * _M1
_KW = _M2
_LF = _KW * _WIDTH

_CPARAMS = pltpu.CompilerParams(dimension_semantics=("parallel",),
                                vmem_limit_bytes=48 * 1024 * 1024)


def _gelu(v):
    return 0.5 * v * (1.0 + jnp.tanh(0.7978845608028654
                                     * (v + 0.044715 * v * v * v)))


def _fc0_kernel(x_ref, w_ref, g_ref, o_ref):
    i = pl.program_id(0)
    y = jnp.dot(x_ref[...], w_ref[...], preferred_element_type=jnp.float32)
    tm = y.shape[0]
    r = lax.broadcasted_iota(jnp.int32, (tm, 1, 1), 0) + i * tm
    j = lax.broadcasted_iota(jnp.int32, (tm, 4, 1), 1)
    gx = ((r & 15) * 4 + j).astype(jnp.float32) * _INV63
    gy = ((r >> 4) & 63).astype(jnp.float32) * _INV63
    g = g_ref[...]
    add = (gx * g[0].reshape(1, 1, 32) + gy * g[1].reshape(1, 1, 32)
           + g[2].reshape(1, 1, 32))
    o_ref[...] = y + add.reshape(tm, 128)


def _res_kernel(x_ref, s_ref, w_ref, b_ref, o_ref):
    y = (s_ref[...]
         + jnp.dot(x_ref[...], w_ref[...], preferred_element_type=jnp.float32)
         + b_ref[...])
    o_ref[...] = _gelu(y)


def _spec_kernel(x_ref, w_ref, o_ref):
    o_ref[...] = jnp.dot(x_ref[...], w_ref[...],
                         preferred_element_type=jnp.float32)


def _fwd_dft_kernel(x_ref, a1_ref, kb_ref, o_ref):
    x = x_ref[...].reshape(_GB * _H, _WC)
    r = jnp.dot(a1_ref[...], x, preferred_element_type=jnp.float32)
    t = jnp.dot(r.astype(jnp.bfloat16), kb_ref[...],
                preferred_element_type=jnp.float32)
    rows = []
    for b in range(_GB):
        top = t[b * 2 * _NK: b * 2 * _NK + _NK]
        bot = t[b * 2 * _NK + _NK: (b + 1) * 2 * _NK]
        rows.append(top[:, :_LF] - bot[:, _LF:])
        rows.append(top[:, _LF:] + bot[:, :_LF])
    o_ref[...] = jnp.concatenate(rows, axis=0).reshape(_GB, 2 * _NK, _LF)


def _inv_dft_kernel(x_ref, a2_ref, ki_ref, o_ref):
    x = x_ref[...].reshape(_GB * 2 * _NK, _LF)
    t = jnp.dot(a2_ref[...], x, preferred_element_type=jnp.float32)
    rows = []
    for b in range(_GB):
        blk = t[b * 2 * _H: (b + 1) * 2 * _H]
        rows.append(jnp.concatenate([blk[:_H], blk[_H:]], axis=1))
    y_in = jnp.concatenate(rows, axis=0)
    y = jnp.dot(y_in.astype(jnp.bfloat16), ki_ref[...],
                preferred_element_type=jnp.float32)
    o_ref[...] = y.reshape(_GB, _H, _WC)


def _tail_kernel(x_ref, s_ref, w3_ref, b3_ref, w1_ref, b1_ref,
                 w2_ref, b2_ref, o_ref):
    h = (s_ref[...]
         + jnp.dot(x_ref[...], w3_ref[...], preferred_element_type=jnp.float32)
         + b3_ref[...])
    t = _gelu(jnp.dot(h, w1_ref[...], preferred_element_type=jnp.float32)
              + b1_ref[...])
    o_ref[...] = (jnp.dot(t, w2_ref[...], preferred_element_type=jnp.float32)
                  + b2_ref[...])


def _row_call(body, in_specs, out_cols, *args):
    return pl.pallas_call(
        body,
        out_shape=jax.ShapeDtypeStruct((_MP, out_cols), jnp.float32),
        grid=(_MP // _TM,),
        in_specs=in_specs,
        out_specs=pl.BlockSpec((_TM, out_cols), lambda i: (i, 0)),
        compiler_params=_CPARAMS,
    )(*args)


def _rowspec(cols):
    return pl.BlockSpec((_TM, cols), lambda i: (i, 0))


def _wspec(rows, cols):
    return pl.BlockSpec((rows, cols), lambda i: (0, 0))


def _dft_consts():
    th = 2.0 * jnp.pi / 64.0
    hh = jnp.arange(64, dtype=jnp.float32)
    kk = jnp.concatenate([jnp.arange(12, dtype=jnp.float32),
                          jnp.arange(52, 64, dtype=jnp.float32)])
    kw = jnp.arange(12, dtype=jnp.float32)
    eye_c = jnp.eye(_WIDTH, dtype=jnp.float32)
    eye_g = jnp.eye(_GB, dtype=jnp.float32)
    a1 = th * kk[:, None] * hh[None, :]
    a1 = jnp.concatenate([jnp.cos(a1), -jnp.sin(a1)], axis=0)
    a1 = jnp.kron(eye_g, a1)
    aw = th * hh[:, None] * kw[None, :]
    kbig = jnp.concatenate([jnp.kron(jnp.cos(aw), eye_c),
                            jnp.kron(-jnp.sin(aw), eye_c)], axis=1)
    a2a = th * hh[:, None] * kk[None, :]
    ar, ai = jnp.cos(a2a) / 64.0, jnp.sin(a2a) / 64.0
    a2 = jnp.concatenate(
        [jnp.concatenate([ar, -ai], axis=1),
         jnp.concatenate([ai, ar], axis=1)], axis=0)
    a2 = jnp.kron(eye_g, a2)
    gk = jnp.where(kw == 0, 1.0, 2.0)
    awi = th * kw[:, None] * hh[None, :]
    kinv = jnp.concatenate(
        [jnp.kron(gk[:, None] * jnp.cos(awi) / 64.0, eye_c),
         jnp.kron(gk[:, None] * -jnp.sin(awi) / 64.0, eye_c)], axis=0)
    return a1, kbig.astype(jnp.bfloat16), a2, kinv.astype(jnp.bfloat16)


def _spectral(h, spec_w, a1, kbig, a2, kinv):
    ng, kg, _ = spec_w.shape
    modes = pl.pallas_call(
        _fwd_dft_kernel,
        out_shape=jax.ShapeDtypeStruct((_B, 2 * _NK, _LF), jnp.float32),
        grid=(_B // _GB,),
        in_specs=[pl.BlockSpec((_GB, _H, _WC), lambda i: (i, 0, 0)),
                  _wspec(*a1.shape), _wspec(*kbig.shape)],
        out_specs=pl.BlockSpec((_GB, 2 * _NK, _LF), lambda i: (i, 0, 0)),
        compiler_params=_CPARAMS,
    )(h.reshape(_B, _H, _WC), a1, kbig)
    re = modes[:, :_NK].reshape(_B, _NK, _KW, _WIDTH)
    im = modes[:, _NK:].reshape(_B, _NK, _KW, _WIDTH)
    xp = jnp.concatenate([re, im], axis=-1).reshape(_B, ng * kg)
    mixed = pl.pallas_call(
        _spec_kernel,
        out_shape=jax.ShapeDtypeStruct((_B, ng * kg), jnp.float32),
        grid=(ng,),
        in_specs=[pl.BlockSpec((_B, kg), lambda i: (0, i)),
                  pl.BlockSpec((None, kg, kg), lambda i: (i, 0, 0))],
        out_specs=pl.BlockSpec((_B, kg), lambda i: (0, i)),
        compiler_params=_CPARAMS,
    )(xp, spec_w)
    mixed = mixed.reshape(_B, _NK * _KW, 2 * _WIDTH)
    rec = mixed[..., :_WIDTH].reshape(_B, _NK, _LF)
    imc = mixed[..., _WIDTH:].reshape(_B, _NK, _LF)
    x3 = jnp.concatenate([rec, imc], axis=1)
    out = pl.pallas_call(
        _inv_dft_kernel,
        out_shape=jax.ShapeDtypeStruct((_B, _H, _WC), jnp.float32),
        grid=(_B // _GB,),
        in_specs=[pl.BlockSpec((_GB, 2 * _NK, _LF), lambda i: (i, 0, 0)),
                  _wspec(*a2.shape), _wspec(*kinv.shape)],
        out_specs=pl.BlockSpec((_GB, _H, _WC), lambda i: (i, 0, 0)),
        compiler_params=_CPARAMS,
    )(x3, a2, kinv)
    return out.reshape(_MP, _PACK * _WIDTH)


def kernel(fc0_w, fc0_b, w0_w, w0_b, spec0_w, w1_w, w1_b, spec1_w,
           w2_w, w2_b, spec2_w, w3_w, w3_b, spec3_w,
           fc1_w, fc1_b, fc2_w, fc2_b, x):
    w16 = fc0_w[:_CIN, :_WIDTH]
    w64 = jnp.kron(jnp.eye(_PACK, dtype=w16.dtype), w16)
    gpar = jnp.stack([fc0_w[_CIN, :_WIDTH], fc0_w[_CIN + 1, :_WIDTH],
                      fc0_b[:_WIDTH], jnp.zeros((_WIDTH,), jnp.float32)])
    a1, kbig, a2, kinv = _dft_consts()

    xp = x.reshape(_MP, _PACK * _CIN)
    h = _row_call(_fc0_kernel,
                  [_rowspec(_PACK * _CIN), _wspec(64, 128),
                   pl.BlockSpec((4, 32), lambda i: (0, 0))],
                  128, xp, w64, gpar)

    for wk, bk, sk in ((w0_w, w0_b, spec0_w), (w1_w, w1_b, spec1_w),
                       (w2_w, w2_b, spec2_w)):
        spec = _spectral(h, sk, a1, kbig, a2, kinv)
        h = _row_call(_res_kernel,
                      [_rowspec(128), _rowspec(128), _wspec(128, 128),
                       pl.BlockSpec((1, 128), lambda i: (0, 0))],
                      128, h, spec, wk, bk.reshape(1, 128))

    spec = _spectral(h, spec3_w, a1, kbig, a2, kinv)
    n1 = fc1_w.shape[1]
    out = _row_call(_tail_kernel,
                    [_rowspec(128), _rowspec(128), _wspec(128, 128),
                     pl.BlockSpec((1, 128), lambda i: (0, 0)),
                     _wspec(128, n1), pl.BlockSpec((1, n1), lambda i: (0, 0)),
                     _wspec(n1, _PACK), pl.BlockSpec((1, _PACK), lambda i: (0, 0))],
                    _PACK, h, spec, w3_w, w3_b.reshape(1, 128),
                    fc1_w, fc1_b.reshape(1, n1), fc2_w, fc2_b.reshape(1, _PACK))
    return out.reshape(_B, _H, _W, 1)
